# baseline (speedup 1.0000x reference)
import jax
import jax.numpy as jnp
from jax.experimental import pallas as pl
from jax.experimental.pallas import tpu as pltpu

_MARGIN = 1.0
_R = 256
_KAUG = 136


def _triplet_block_kernel(eall_ref, erow_ref, labcand_ref, labanc_ref,
                          tot_ref, cnt_ref, dist_ref):
    i = pl.program_id(0)

    rhs = erow_ref[...]
    d2 = jax.lax.dot_general(
        eall_ref[...], rhs, (((1,), (1,)), ((), ())),
        preferred_element_type=jnp.float32)
    same = labcand_ref[...] == labanc_ref[0:1, :]
    dist_ref[...] = jnp.where(same, d2, -jnp.inf)
    neg = jnp.where(same, jnp.inf, d2)
    hn = jnp.maximum(jnp.min(neg, axis=0, keepdims=True), 0.0)

    rr = jax.lax.broadcasted_iota(jnp.int32, (_R, _R), 0)
    cc = jax.lax.broadcasted_iota(jnp.int32, (_R, _R), 1)
    blk = dist_ref[pl.ds(i * _R, _R), :]
    dist_ref[pl.ds(i * _R, _R), :] = jnp.where(rr == cc, -jnp.inf, blk)

    d = dist_ref[...]
    g = d - hn
    p = jnp.maximum(g, 0.0)
    ind = jnp.where(p > 0.0, 1.0, 0.0)
    tot_ref[0, :, :] = jnp.sum(p, axis=0, keepdims=True)
    cnt_ref[0, :, :] = jnp.sum(ind, axis=0, keepdims=True)


def kernel(embeddings, labels):
    e = embeddings.astype(jnp.float32)
    B, D = e.shape
    lab = labels.astype(jnp.int32)
    nb = B // _R

    sq = jnp.sum(e * e, axis=1)
    hi = sq.astype(jnp.bfloat16).astype(jnp.float32)
    lo = sq - hi
    one = jnp.ones((B, 1), jnp.float32)
    zed = jnp.zeros((B, 4), jnp.float32)
    eaug_a = jnp.concatenate(
        [-2.0 * e, hi[:, None], lo[:, None], one, one, zed],
        axis=1).astype(jnp.bfloat16)
    eaug_b = jnp.concatenate(
        [e, one, one, hi[:, None], lo[:, None], zed],
        axis=1).astype(jnp.bfloat16)
    labcand = lab.reshape(B, 1)
    labanc = lab.reshape(1, B)

    tot_parts, cnt_parts = pl.pallas_call(
        _triplet_block_kernel,
        grid=(nb,),
        in_specs=[
            pl.BlockSpec((B, _KAUG), lambda i: (0, 0)),
            pl.BlockSpec((_R, _KAUG), lambda i: (i, 0)),
            pl.BlockSpec((B, 1), lambda i: (0, 0)),
            pl.BlockSpec((1, _R), lambda i: (0, i)),
        ],
        out_specs=[
            pl.BlockSpec((1, 1, _R), lambda i: (i, 0, 0)),
            pl.BlockSpec((1, 1, _R), lambda i: (i, 0, 0)),
        ],
        out_shape=[
            jax.ShapeDtypeStruct((nb, 1, _R), jnp.float32),
            jax.ShapeDtypeStruct((nb, 1, _R), jnp.float32),
        ],
        scratch_shapes=[pltpu.VMEM((B, _R), jnp.float32)],
        compiler_params=pltpu.CompilerParams(
            dimension_semantics=("parallel",),
            vmem_limit_bytes=48 * 1024 * 1024,
        ),
        name="triplet_mining",
    )(eaug_a, eaug_b, labcand, labanc)

    count = jnp.sum(cnt_parts.astype(jnp.int32))
    total = jnp.sum(tot_parts) + count.astype(jnp.float32)
    loss = total / jnp.maximum(count, 1).astype(jnp.float32)
    return loss, count


# speedup vs baseline: 1.1323x; 1.1323x over previous
import jax
import jax.numpy as jnp
from jax.experimental import pallas as pl
from jax.experimental.pallas import tpu as pltpu

_MARGIN = 1.0
_R = 256
_KAUX = 8


def _triplet_block_kernel(eall_ref, erow_ref, auxa_ref, auxb_ref,
                          labcand_ref, labanc_ref, tot_ref, cnt_ref, h_ref):
    i = pl.program_id(0)

    h = jax.lax.dot_general(
        eall_ref[...], erow_ref[...], (((1,), (1,)), ((), ())),
        preferred_element_type=jnp.float32)
    h = h + jax.lax.dot_general(
        auxa_ref[...], auxb_ref[...], (((1,), (1,)), ((), ())),
        preferred_element_type=jnp.float32)
    same = labcand_ref[...] == labanc_ref[0:1, :]
    h_ref[...] = jnp.where(same, h, jnp.inf)
    neg = jnp.where(same, -jnp.inf, h)
    hn = jnp.minimum(jnp.max(neg, axis=0, keepdims=True), 0.0)

    rr = jax.lax.broadcasted_iota(jnp.int32, (_R, _R), 0)
    cc = jax.lax.broadcasted_iota(jnp.int32, (_R, _R), 1)
    blk = h_ref[pl.ds(i * _R, _R), :]
    h_ref[pl.ds(i * _R, _R), :] = jnp.where(rr == cc, jnp.inf, blk)

    hp = h_ref[...]
    ph = jnp.maximum(hn - hp, 0.0)
    ind = jnp.where(ph > 0.0, 1.0, 0.0)
    tot_ref[0, :, :] = jnp.sum(ph, axis=0, keepdims=True)
    cnt_ref[0, :, :] = jnp.sum(ind, axis=0, keepdims=True)


def kernel(embeddings, labels):
    e = embeddings.astype(jnp.float32)
    B, D = e.shape
    lab = labels.astype(jnp.int32)
    nb = B // _R

    eb = e.astype(jnp.bfloat16)
    s2 = -0.5 * jnp.sum(e * e, axis=1, keepdims=True)
    hi = s2.astype(jnp.bfloat16).astype(jnp.float32)
    lo = s2 - hi
    one = jnp.ones((B, 1), jnp.float32)
    zed = jnp.zeros((B, 4), jnp.float32)
    aux_a = jnp.concatenate([hi, lo, one, one, zed],
                            axis=1).astype(jnp.bfloat16)
    aux_b = jnp.concatenate([one, one, hi, lo, zed],
                            axis=1).astype(jnp.bfloat16)
    labcand = lab.reshape(B, 1)
    labanc = lab.reshape(1, B)

    tot_parts, cnt_parts = pl.pallas_call(
        _triplet_block_kernel,
        grid=(nb,),
        in_specs=[
            pl.BlockSpec((B, D), lambda i: (0, 0)),
            pl.BlockSpec((_R, D), lambda i: (i, 0)),
            pl.BlockSpec((B, _KAUX), lambda i: (0, 0)),
            pl.BlockSpec((_R, _KAUX), lambda i: (i, 0)),
            pl.BlockSpec((B, 1), lambda i: (0, 0)),
            pl.BlockSpec((1, _R), lambda i: (0, i)),
        ],
        out_specs=[
            pl.BlockSpec((1, 1, _R), lambda i: (i, 0, 0)),
            pl.BlockSpec((1, 1, _R), lambda i: (i, 0, 0)),
        ],
        out_shape=[
            jax.ShapeDtypeStruct((nb, 1, _R), jnp.float32),
            jax.ShapeDtypeStruct((nb, 1, _R), jnp.float32),
        ],
        scratch_shapes=[pltpu.VMEM((B, _R), jnp.float32)],
        compiler_params=pltpu.CompilerParams(
            dimension_semantics=("parallel",),
            vmem_limit_bytes=48 * 1024 * 1024,
        ),
        name="triplet_mining",
    )(eb, eb, aux_a, aux_b, labcand, labanc)

    count = jnp.sum(cnt_parts.astype(jnp.int32))
    total = 2.0 * jnp.sum(tot_parts) + count.astype(jnp.float32)
    loss = total / jnp.maximum(count, 1).astype(jnp.float32)
    return loss, count
